# baseline (speedup 1.0000x reference)
import jax
import jax.numpy as jnp
from jax import lax
from jax.experimental import pallas as pl
from jax.experimental.pallas import tpu as pltpu
from jax.experimental.pallas import tpu_sc as plsc

NUM_EXPERTS = 8
D_MODEL = 2048
D_HALF = D_MODEL // 2
D_FF = 1024
SEQ = 2048
NPAIR = 2 * SEQ

TBLK = 256
NT = SEQ // TBLK
BT = 128
PMAX = NPAIR + NUM_EXPERTS * BT
GMAX = PMAX // BT
MOFF = 8
MNACT = MOFF + 64

NC, NS, L = 2, 16, 16
NW = NC * NS

_F32 = jnp.float32
_I32 = jnp.int32
_HIMASK = -65536


def _router_body(x_ref, gw_ref, logits_ref, eids_ref, rank_ref, sb_ref,
                 meta_ref, cnt_scr):
    t = pl.program_id(0)

    @pl.when(t == 0)
    def _init():
        cnt_scr[...] = jnp.zeros_like(cnt_scr)

    x = x_ref[...]
    logits = lax.dot_general(x, gw_ref[...], (((1,), (1,)), ((), ())),
                             preferred_element_type=_F32)
    logits_ref[...] = logits

    idx = lax.broadcasted_iota(_I32, (TBLK, NUM_EXPERTS), 1)
    m1 = jnp.max(logits, axis=-1, keepdims=True)
    e1 = jnp.min(jnp.where(logits == m1, idx, NUM_EXPERTS), axis=-1,
                 keepdims=True)
    lm = jnp.where(idx == e1, -jnp.inf, logits)
    m2 = jnp.max(lm, axis=-1, keepdims=True)
    e2 = jnp.min(jnp.where(lm == m2, idx, NUM_EXPERTS), axis=-1,
                 keepdims=True)
    w1 = 1.0 / (1.0 + jnp.exp(m2 - m1))
    w2 = 1.0 - w1

    eids_ref[...] = jnp.concatenate([e1, e2], axis=1)
    sb_ref[...] = jnp.concatenate(
        [jnp.broadcast_to(w1, (TBLK, L)), jnp.broadcast_to(w2, (TBLK, L))],
        axis=1)

    oh1 = jnp.where(idx == e1, 1.0, 0.0)
    oh2 = jnp.where(idx == e2, 1.0, 0.0)
    s_blk = oh1 + oh2

    r_io = lax.broadcasted_iota(_I32, (TBLK, TBLK), 0)
    c_io = lax.broadcasted_iota(_I32, (TBLK, TBLK), 1)
    lstrict = jnp.where(c_io < r_io, 1.0, 0.0)
    e_blk = lax.dot_general(lstrict, s_blk, (((1,), (0,)), ((), ())),
                            preferred_element_type=_F32) + cnt_scr[...]
    rank1 = jnp.sum(oh1 * e_blk, axis=-1, keepdims=True)
    rank2 = jnp.sum(oh2 * e_blk, axis=-1, keepdims=True)
    rank_ref[...] = jnp.concatenate([rank1, rank2], axis=1).astype(_I32)

    new_cnt = cnt_scr[...] + jnp.sum(s_blk, axis=0, keepdims=True)
    cnt_scr[...] = new_cnt

    @pl.when(t == NT - 1)
    def _meta():
        row_io = lax.broadcasted_iota(_I32, (8, 8), 0)
        col_io = lax.broadcasted_iota(_I32, (8, 8), 1)
        cnt_bc = jnp.broadcast_to(new_cnt, (8, 8))
        padded_bc = jnp.floor((cnt_bc + (BT - 1)) / BT) * BT
        cum_col = jnp.sum(jnp.where(col_io <= row_io, padded_bc, 0.0),
                          axis=1, keepdims=True)
        padded_col = jnp.sum(jnp.where(col_io == row_io, padded_bc, 0.0),
                             axis=1, keepdims=True)
        blocks_cum = cum_col / BT
        g_io = lax.broadcasted_iota(_I32, (8, 64), 1).astype(_F32)
        be_row = jnp.minimum(
            jnp.sum(jnp.where(g_io >= blocks_cum, 1.0, 0.0), axis=0,
                    keepdims=True), 7.0)
        padded_lbc = jnp.broadcast_to(padded_col, (8, 8))
        off_row = jnp.sum(jnp.where(row_io < col_io, padded_lbc, 0.0),
                          axis=0, keepdims=True)
        nact = jnp.sum(padded_col, axis=0, keepdims=True) / BT
        pad = jnp.zeros((1, 128 - MNACT - 1), _F32)
        meta_ref[...] = jnp.concatenate([off_row, be_row, nact, pad],
                                        axis=1).astype(_I32)


def _router(x, gate_w):
    return pl.pallas_call(
        _router_body,
        grid=(NT,),
        in_specs=[
            pl.BlockSpec((TBLK, D_MODEL), lambda t: (t, 0)),
            pl.BlockSpec((NUM_EXPERTS, D_MODEL), lambda t: (0, 0)),
        ],
        out_specs=[
            pl.BlockSpec((TBLK, NUM_EXPERTS), lambda t: (t, 0)),
            pl.BlockSpec((TBLK, 2), lambda t: (t, 0)),
            pl.BlockSpec((TBLK, 2), lambda t: (t, 0)),
            pl.BlockSpec((TBLK, 2 * L), lambda t: (t, 0)),
            pl.BlockSpec((1, 128), lambda t: (0, 0)),
        ],
        out_shape=[
            jax.ShapeDtypeStruct((SEQ, NUM_EXPERTS), _F32),
            jax.ShapeDtypeStruct((SEQ, 2), _I32),
            jax.ShapeDtypeStruct((SEQ, 2), _I32),
            jax.ShapeDtypeStruct((SEQ, 2 * L), _F32),
            jax.ShapeDtypeStruct((1, 128), _I32),
        ],
        scratch_shapes=[pltpu.VMEM((1, NUM_EXPERTS), _F32)],
        compiler_params=pltpu.CompilerParams(
            dimension_semantics=("arbitrary",)),
    )(x, gate_w)


def _sc_mesh():
    return plsc.VectorSubcoreMesh(core_axis_name="c", subcore_axis_name="s",
                                  num_cores=NC, num_subcores=NS)


def _dispatch_body(eids_hbm, rank_hbm, meta_hbm, pos_hbm, tok_hbm,
                   e_v, r_v, meta_v, pos_v, tok_v):
    wid = lax.axis_index("s") * NC + lax.axis_index("c")

    @pl.when(wid == 0)
    def _work():
        pltpu.sync_copy(eids_hbm, e_v)
        pltpu.sync_copy(rank_hbm, r_v)
        pltpu.sync_copy(meta_hbm, meta_v)

        @pl.loop(0, PMAX // L)
        def _zero(i):
            tok_v[pl.ds(i * L, L)] = jnp.zeros((L,), _I32)

        @pl.loop(0, NPAIR // L)
        def _scan(i):
            sl = pl.ds(i * L, L)
            e = e_v[sl]
            r = r_v[sl]
            off_e = plsc.load_gather(meta_v, [e])
            p = off_e + r
            pos_v[sl] = p
            tok = (i * L + lax.iota(_I32, L)) >> 1
            plsc.store_scatter(tok_v, [p], tok)

        pltpu.sync_copy(pos_v, pos_hbm)
        pltpu.sync_copy(tok_v, tok_hbm)


def _dispatch(eids, rank, meta):
    return pl.kernel(
        _dispatch_body,
        out_type=[
            jax.ShapeDtypeStruct((NPAIR,), _I32),
            jax.ShapeDtypeStruct((PMAX,), _I32),
        ],
        mesh=_sc_mesh(),
        compiler_params=pltpu.CompilerParams(needs_layout_passes=False),
        scratch_types=[
            pltpu.VMEM((NPAIR,), _I32),
            pltpu.VMEM((NPAIR,), _I32),
            pltpu.VMEM((128,), _I32),
            pltpu.VMEM((NPAIR,), _I32),
            pltpu.VMEM((PMAX,), _I32),
        ],
    )(eids, rank, meta)


_GCH = 16
_GPW = PMAX // NW
_GNCH = _GPW // _GCH
_GRING = 4


def _gather_body(tok_hbm, x_hbm, xs_hbm, idx_v, rows_r, g_sem, w_sem):
    wid = lax.axis_index("s") * NC + lax.axis_index("c")
    wbase = wid * _GPW
    pltpu.sync_copy(tok_hbm.at[pl.ds(wbase, _GPW)], idx_v)

    def _start_gather(c):
        return pltpu.async_copy(
            x_hbm.at[idx_v.at[pl.ds(c * _GCH, _GCH)]],
            rows_r.at[c % _GRING], g_sem)

    gd = [None] * _GNCH
    wd = [None] * _GNCH
    gd[0] = _start_gather(0)
    for c in range(_GNCH):
        if c + 1 < _GNCH:
            if c + 1 >= _GRING:
                wd[c + 1 - _GRING].wait()
            gd[c + 1] = _start_gather(c + 1)
        gd[c].wait()
        wd[c] = pltpu.async_copy(
            rows_r.at[c % _GRING], xs_hbm.at[pl.ds(wbase + c * _GCH, _GCH)],
            w_sem)
    for c in range(max(0, _GNCH - _GRING), _GNCH):
        wd[c].wait()


def _gather(tok, x_p):
    return pl.kernel(
        _gather_body,
        out_type=jax.ShapeDtypeStruct((PMAX, D_HALF), _I32),
        mesh=_sc_mesh(),
        compiler_params=pltpu.CompilerParams(needs_layout_passes=False),
        scratch_types=[
            pltpu.VMEM((_GPW,), _I32),
            pltpu.VMEM((_GRING, _GCH, D_HALF), _I32),
            pltpu.SemaphoreType.DMA,
            pltpu.SemaphoreType.DMA,
        ],
    )(tok, x_p)


def _ffn_body(meta_sref, xs_ref, wg_ref, wu_ref, wd_ref, ys_ref):
    g = pl.program_id(0)

    @pl.when(g < meta_sref[MNACT])
    def _active():
        xs = xs_ref[...]
        xlo = lax.bitcast_convert_type(xs << 16, _F32).astype(jnp.bfloat16)
        xhi = lax.bitcast_convert_type(xs & _HIMASK, _F32).astype(jnp.bfloat16)
        wg = wg_ref[0]
        wu = wu_ref[0]
        wd = wd_ref[0]
        dims = (((1,), (1,)), ((), ()))
        gg = (lax.dot_general(xlo, wg[:, :D_HALF], dims,
                              preferred_element_type=_F32)
              + lax.dot_general(xhi, wg[:, D_HALF:], dims,
                                preferred_element_type=_F32))
        uu = (lax.dot_general(xlo, wu[:, :D_HALF], dims,
                              preferred_element_type=_F32)
              + lax.dot_general(xhi, wu[:, D_HALF:], dims,
                                preferred_element_type=_F32))
        h = (gg * jax.nn.sigmoid(gg) * uu).astype(jnp.bfloat16)
        y = lax.dot_general(h, wd, dims, preferred_element_type=_F32)
        ylo = lax.bitcast_convert_type(
            y[:, :D_HALF].astype(jnp.bfloat16), jnp.uint16).astype(jnp.uint32)
        yhi = lax.bitcast_convert_type(
            y[:, D_HALF:].astype(jnp.bfloat16), jnp.uint16).astype(jnp.uint32)
        ys_ref[...] = lax.bitcast_convert_type(ylo | (yhi << 16), _I32)


def _gclamp(g, m):
    return jnp.minimum(g, m[MNACT] - 1)


def _ffn(meta_flat, xs, wg, wu, wd):
    grid_spec = pltpu.PrefetchScalarGridSpec(
        num_scalar_prefetch=1,
        grid=(GMAX,),
        in_specs=[
            pl.BlockSpec((BT, D_HALF), lambda g, m: (_gclamp(g, m), 0)),
            pl.BlockSpec((1, D_FF, D_MODEL),
                         lambda g, m: (m[MOFF + _gclamp(g, m)], 0, 0)),
            pl.BlockSpec((1, D_FF, D_MODEL),
                         lambda g, m: (m[MOFF + _gclamp(g, m)], 0, 0)),
            pl.BlockSpec((1, D_MODEL, D_FF),
                         lambda g, m: (m[MOFF + _gclamp(g, m)], 0, 0)),
        ],
        out_specs=pl.BlockSpec((BT, D_HALF), lambda g, m: (_gclamp(g, m), 0)),
    )
    return pl.pallas_call(
        _ffn_body,
        grid_spec=grid_spec,
        out_shape=jax.ShapeDtypeStruct((PMAX, D_HALF), _I32),
        compiler_params=pltpu.CompilerParams(
            dimension_semantics=("arbitrary",)),
    )(meta_flat, xs, wg, wu, wd)


_CCH = 8
_CPW = SEQ // NW
_CNCH = _CPW // _CCH


def _combine_body(pos_hbm, sb_hbm, ys_hbm, out_hbm,
                  idx_v, sb_v, rows2, out2, g_sem, w_sem):
    wid = lax.axis_index("s") * NC + lax.axis_index("c")
    tbase = wid * _CPW
    pltpu.sync_copy(pos_hbm.at[pl.ds(2 * tbase, 2 * _CPW)], idx_v)
    pltpu.sync_copy(sb_hbm.at[pl.ds(2 * tbase, 2 * _CPW)], sb_v)

    def _start_gather(c):
        return pltpu.async_copy(
            ys_hbm.at[idx_v.at[pl.ds(c * 2 * _CCH, 2 * _CCH)]],
            rows2.at[c % 2], g_sem)

    gd = [None] * _CNCH
    wd = [None] * _CNCH
    gd[0] = _start_gather(0)
    for c in range(_CNCH):
        if c + 1 < _CNCH:
            gd[c + 1] = _start_gather(c + 1)
        gd[c].wait()
        if c >= 2:
            wd[c - 2].wait()
        rows_v = rows2.at[c % 2]
        out_v = out2.at[c % 2]

        @pl.loop(0, _CCH)
        def _tok(i):
            p = c * 2 * _CCH + 2 * i
            s0 = sb_v[p, :]
            s1 = sb_v[p + 1, :]
            for j in range(D_HALF // L):
                sl = pl.ds(j * L, L)
                a0 = rows_v[2 * i, sl]
                a1 = rows_v[2 * i + 1, sl]
                lo0 = plsc.bitcast(a0 << 16, _F32)
                lo1 = plsc.bitcast(a1 << 16, _F32)
                hi0 = plsc.bitcast(a0 & _HIMASK, _F32)
                hi1 = plsc.bitcast(a1 & _HIMASK, _F32)
                out_v[i, sl] = s0 * lo0 + s1 * lo1
                out_v[i, pl.ds(D_HALF + j * L, L)] = s0 * hi0 + s1 * hi1

        wd[c] = pltpu.async_copy(
            out_v, out_hbm.at[pl.ds(tbase + c * _CCH, _CCH)], w_sem)
    for c in range(max(0, _CNCH - 2), _CNCH):
        wd[c].wait()


def _combine(pos, sb, ys):
    return pl.kernel(
        _combine_body,
        out_type=jax.ShapeDtypeStruct((SEQ, D_MODEL), _F32),
        mesh=_sc_mesh(),
        compiler_params=pltpu.CompilerParams(needs_layout_passes=False),
        scratch_types=[
            pltpu.VMEM((2 * _CPW,), _I32),
            pltpu.VMEM((2 * _CPW, L), _F32),
            pltpu.VMEM((2, 2 * _CCH, D_HALF), _I32),
            pltpu.VMEM((2, _CCH, D_MODEL), _F32),
            pltpu.SemaphoreType.DMA,
            pltpu.SemaphoreType.DMA,
        ],
    )(pos, sb, ys)


def kernel(hidden_states, gate_w, w_gate, w_up, w_down):
    B, S, H = hidden_states.shape
    x = hidden_states.reshape(S, H)
    be = lax.bitcast_convert_type(
        x[:, :D_HALF].astype(jnp.bfloat16), jnp.uint16).astype(jnp.uint32)
    bo = lax.bitcast_convert_type(
        x[:, D_HALF:].astype(jnp.bfloat16), jnp.uint16).astype(jnp.uint32)
    x_p = lax.bitcast_convert_type(be | (bo << 16), _I32)

    logits, eids, rank, sb, meta = _router(x, gate_w)
    meta_flat = meta.reshape(128)
    pos, tok = _dispatch(eids.reshape(NPAIR), rank.reshape(NPAIR), meta_flat)
    xs = _gather(tok, x_p)
    dep = (tok[0] * 0).astype(_F32)
    wgb = (w_gate + dep).astype(jnp.bfloat16)
    wub = (w_up + dep).astype(jnp.bfloat16)
    wdb = (w_down + dep).astype(jnp.bfloat16)
    ys = _ffn(meta_flat, xs, wgb, wub, wdb)
    out = _combine(pos, sb.reshape(NPAIR, L), ys)
    return out.reshape(B, S, H), logits


# speedup vs baseline: 1.2764x; 1.2764x over previous
import jax
import jax.numpy as jnp
from jax import lax
from jax.experimental import pallas as pl
from jax.experimental.pallas import tpu as pltpu
from jax.experimental.pallas import tpu_sc as plsc

NUM_EXPERTS = 8
D_MODEL = 2048
D_HALF = D_MODEL // 2
D_FF = 1024
SEQ = 2048
NPAIR = 2 * SEQ

TBLK = 256
NT = SEQ // TBLK
BT = 256
PMAX = NPAIR + NUM_EXPERTS * BT
GMAX = PMAX // BT
MOFF = 8
MNACT = MOFF + 64

NC, NS, L = 2, 16, 16
NW = NC * NS

_F32 = jnp.float32
_I32 = jnp.int32
_HIMASK = -65536


def _router_body(x_ref, gw_ref, logits_ref, eids_ref, rank_ref, sb_ref,
                 meta_ref, cnt_scr):
    t = pl.program_id(0)

    @pl.when(t == 0)
    def _init():
        cnt_scr[...] = jnp.zeros_like(cnt_scr)

    x = x_ref[...]
    logits = lax.dot_general(x, gw_ref[...], (((1,), (1,)), ((), ())),
                             preferred_element_type=_F32)
    logits_ref[...] = logits

    idx = lax.broadcasted_iota(_I32, (TBLK, NUM_EXPERTS), 1)
    m1 = jnp.max(logits, axis=-1, keepdims=True)
    e1 = jnp.min(jnp.where(logits == m1, idx, NUM_EXPERTS), axis=-1,
                 keepdims=True)
    lm = jnp.where(idx == e1, -jnp.inf, logits)
    m2 = jnp.max(lm, axis=-1, keepdims=True)
    e2 = jnp.min(jnp.where(lm == m2, idx, NUM_EXPERTS), axis=-1,
                 keepdims=True)
    w1 = 1.0 / (1.0 + jnp.exp(m2 - m1))
    w2 = 1.0 - w1

    eids_ref[...] = jnp.concatenate([e1, e2], axis=1)
    sb_ref[...] = jnp.concatenate(
        [jnp.broadcast_to(w1, (TBLK, L)), jnp.broadcast_to(w2, (TBLK, L))],
        axis=1)

    oh1 = jnp.where(idx == e1, 1.0, 0.0)
    oh2 = jnp.where(idx == e2, 1.0, 0.0)
    s_blk = oh1 + oh2

    r_io = lax.broadcasted_iota(_I32, (TBLK, TBLK), 0)
    c_io = lax.broadcasted_iota(_I32, (TBLK, TBLK), 1)
    lstrict = jnp.where(c_io < r_io, 1.0, 0.0)
    e_blk = lax.dot_general(lstrict, s_blk, (((1,), (0,)), ((), ())),
                            preferred_element_type=_F32) + cnt_scr[...]
    rank1 = jnp.sum(oh1 * e_blk, axis=-1, keepdims=True)
    rank2 = jnp.sum(oh2 * e_blk, axis=-1, keepdims=True)
    rank_ref[...] = jnp.concatenate([rank1, rank2], axis=1).astype(_I32)

    new_cnt = cnt_scr[...] + jnp.sum(s_blk, axis=0, keepdims=True)
    cnt_scr[...] = new_cnt

    @pl.when(t == NT - 1)
    def _meta():
        row_io = lax.broadcasted_iota(_I32, (8, 8), 0)
        col_io = lax.broadcasted_iota(_I32, (8, 8), 1)
        cnt_bc = jnp.broadcast_to(new_cnt, (8, 8))
        padded_bc = jnp.floor((cnt_bc + (BT - 1)) / BT) * BT
        cum_col = jnp.sum(jnp.where(col_io <= row_io, padded_bc, 0.0),
                          axis=1, keepdims=True)
        padded_col = jnp.sum(jnp.where(col_io == row_io, padded_bc, 0.0),
                             axis=1, keepdims=True)
        blocks_cum = cum_col / BT
        g_io = lax.broadcasted_iota(_I32, (8, 64), 1).astype(_F32)
        be_row = jnp.minimum(
            jnp.sum(jnp.where(g_io >= blocks_cum, 1.0, 0.0), axis=0,
                    keepdims=True), 7.0)
        padded_lbc = jnp.broadcast_to(padded_col, (8, 8))
        off_row = jnp.sum(jnp.where(row_io < col_io, padded_lbc, 0.0),
                          axis=0, keepdims=True)
        nact = jnp.sum(padded_col, axis=0, keepdims=True) / BT
        pad = jnp.zeros((1, 128 - MNACT - 1), _F32)
        meta_ref[...] = jnp.concatenate([off_row, be_row, nact, pad],
                                        axis=1).astype(_I32)


def _router(x, gate_w):
    return pl.pallas_call(
        _router_body,
        grid=(NT,),
        in_specs=[
            pl.BlockSpec((TBLK, D_MODEL), lambda t: (t, 0)),
            pl.BlockSpec((NUM_EXPERTS, D_MODEL), lambda t: (0, 0)),
        ],
        out_specs=[
            pl.BlockSpec((TBLK, NUM_EXPERTS), lambda t: (t, 0)),
            pl.BlockSpec((TBLK, 2), lambda t: (t, 0)),
            pl.BlockSpec((TBLK, 2), lambda t: (t, 0)),
            pl.BlockSpec((TBLK, 2 * L), lambda t: (t, 0)),
            pl.BlockSpec((1, 128), lambda t: (0, 0)),
        ],
        out_shape=[
            jax.ShapeDtypeStruct((SEQ, NUM_EXPERTS), _F32),
            jax.ShapeDtypeStruct((SEQ, 2), _I32),
            jax.ShapeDtypeStruct((SEQ, 2), _I32),
            jax.ShapeDtypeStruct((SEQ, 2 * L), _F32),
            jax.ShapeDtypeStruct((1, 128), _I32),
        ],
        scratch_shapes=[pltpu.VMEM((1, NUM_EXPERTS), _F32)],
        compiler_params=pltpu.CompilerParams(
            dimension_semantics=("arbitrary",)),
    )(x, gate_w)


def _sc_mesh():
    return plsc.VectorSubcoreMesh(core_axis_name="c", subcore_axis_name="s",
                                  num_cores=NC, num_subcores=NS)


def _dispatch_body(eids_hbm, rank_hbm, meta_hbm, pos_hbm, tok_hbm,
                   e_v, r_v, meta_v, pos_v, tok_v):
    wid = lax.axis_index("s") * NC + lax.axis_index("c")

    @pl.when(wid == 0)
    def _work():
        pltpu.sync_copy(eids_hbm, e_v)
        pltpu.sync_copy(rank_hbm, r_v)
        pltpu.sync_copy(meta_hbm, meta_v)

        @pl.loop(0, PMAX // L)
        def _zero(i):
            tok_v[pl.ds(i * L, L)] = jnp.zeros((L,), _I32)

        @pl.loop(0, NPAIR // L)
        def _scan(i):
            sl = pl.ds(i * L, L)
            e = e_v[sl]
            r = r_v[sl]
            off_e = plsc.load_gather(meta_v, [e])
            p = off_e + r
            pos_v[sl] = p
            tok = (i * L + lax.iota(_I32, L)) >> 1
            plsc.store_scatter(tok_v, [p], tok)

        pltpu.sync_copy(pos_v, pos_hbm)
        pltpu.sync_copy(tok_v, tok_hbm)


def _dispatch(eids, rank, meta):
    return pl.kernel(
        _dispatch_body,
        out_type=[
            jax.ShapeDtypeStruct((NPAIR,), _I32),
            jax.ShapeDtypeStruct((PMAX,), _I32),
        ],
        mesh=_sc_mesh(),
        compiler_params=pltpu.CompilerParams(needs_layout_passes=False),
        scratch_types=[
            pltpu.VMEM((NPAIR,), _I32),
            pltpu.VMEM((NPAIR,), _I32),
            pltpu.VMEM((128,), _I32),
            pltpu.VMEM((NPAIR,), _I32),
            pltpu.VMEM((PMAX,), _I32),
        ],
    )(eids, rank, meta)


_GCH = 16
_GPW = PMAX // NW
_GNCH = _GPW // _GCH
_GRING = 4


def _gather_body(tok_hbm, x_hbm, xs_hbm, idx_v, rows_r, g_sem, w_sem):
    wid = lax.axis_index("s") * NC + lax.axis_index("c")
    wbase = wid * _GPW
    pltpu.sync_copy(tok_hbm.at[pl.ds(wbase, _GPW)], idx_v)

    def _start_gather(c):
        return pltpu.async_copy(
            x_hbm.at[idx_v.at[pl.ds(c * _GCH, _GCH)]],
            rows_r.at[c % _GRING], g_sem)

    gd = [None] * _GNCH
    wd = [None] * _GNCH
    gd[0] = _start_gather(0)
    for c in range(_GNCH):
        if c + 1 < _GNCH:
            if c + 1 >= _GRING:
                wd[c + 1 - _GRING].wait()
            gd[c + 1] = _start_gather(c + 1)
        gd[c].wait()
        wd[c] = pltpu.async_copy(
            rows_r.at[c % _GRING], xs_hbm.at[pl.ds(wbase + c * _GCH, _GCH)],
            w_sem)
    for c in range(max(0, _GNCH - _GRING), _GNCH):
        wd[c].wait()


def _gather(tok, x_p):
    return pl.kernel(
        _gather_body,
        out_type=jax.ShapeDtypeStruct((PMAX, D_HALF), _I32),
        mesh=_sc_mesh(),
        compiler_params=pltpu.CompilerParams(needs_layout_passes=False),
        scratch_types=[
            pltpu.VMEM((_GPW,), _I32),
            pltpu.VMEM((_GRING, _GCH, D_HALF), _I32),
            pltpu.SemaphoreType.DMA,
            pltpu.SemaphoreType.DMA,
        ],
    )(tok, x_p)


def _ffn_body(meta_sref, xs_ref, wg_ref, wu_ref, wd_ref, ys_ref):
    g = pl.program_id(0)

    @pl.when(g < meta_sref[MNACT])
    def _active():
        xs = xs_ref[...]
        xlo = lax.bitcast_convert_type(xs << 16, _F32).astype(jnp.bfloat16)
        xhi = lax.bitcast_convert_type(xs & _HIMASK, _F32).astype(jnp.bfloat16)
        wg = wg_ref[0].astype(jnp.bfloat16)
        wu = wu_ref[0].astype(jnp.bfloat16)
        wd = wd_ref[0].astype(jnp.bfloat16)
        dims = (((1,), (1,)), ((), ()))
        gg = (lax.dot_general(xlo, wg[:, :D_HALF], dims,
                              preferred_element_type=_F32)
              + lax.dot_general(xhi, wg[:, D_HALF:], dims,
                                preferred_element_type=_F32))
        uu = (lax.dot_general(xlo, wu[:, :D_HALF], dims,
                              preferred_element_type=_F32)
              + lax.dot_general(xhi, wu[:, D_HALF:], dims,
                                preferred_element_type=_F32))
        h = (gg * jax.nn.sigmoid(gg) * uu).astype(jnp.bfloat16)
        y = lax.dot_general(h, wd, dims, preferred_element_type=_F32)
        ylo = lax.bitcast_convert_type(
            y[:, :D_HALF].astype(jnp.bfloat16), jnp.uint16).astype(jnp.uint32)
        yhi = lax.bitcast_convert_type(
            y[:, D_HALF:].astype(jnp.bfloat16), jnp.uint16).astype(jnp.uint32)
        ys_ref[...] = lax.bitcast_convert_type(ylo | (yhi << 16), _I32)


def _gclamp(g, m):
    return jnp.minimum(g, m[MNACT] - 1)


def _ffn(meta_flat, xs, wg, wu, wd):
    grid_spec = pltpu.PrefetchScalarGridSpec(
        num_scalar_prefetch=1,
        grid=(GMAX,),
        in_specs=[
            pl.BlockSpec((BT, D_HALF), lambda g, m: (_gclamp(g, m), 0)),
            pl.BlockSpec((1, D_FF, D_MODEL),
                         lambda g, m: (m[MOFF + _gclamp(g, m)], 0, 0)),
            pl.BlockSpec((1, D_FF, D_MODEL),
                         lambda g, m: (m[MOFF + _gclamp(g, m)], 0, 0)),
            pl.BlockSpec((1, D_MODEL, D_FF),
                         lambda g, m: (m[MOFF + _gclamp(g, m)], 0, 0)),
        ],
        out_specs=pl.BlockSpec((BT, D_HALF), lambda g, m: (_gclamp(g, m), 0)),
    )
    return pl.pallas_call(
        _ffn_body,
        grid_spec=grid_spec,
        out_shape=jax.ShapeDtypeStruct((PMAX, D_HALF), _I32),
        compiler_params=pltpu.CompilerParams(
            dimension_semantics=("arbitrary",)),
    )(meta_flat, xs, wg, wu, wd)


_CCH = 8
_CPW = SEQ // NW
_CNCH = _CPW // _CCH


def _combine_body(pos_hbm, sb_hbm, ys_hbm, out_hbm,
                  idx_v, sb_v, rows2, out2, g_sem, w_sem):
    wid = lax.axis_index("s") * NC + lax.axis_index("c")
    tbase = wid * _CPW
    pltpu.sync_copy(pos_hbm.at[pl.ds(2 * tbase, 2 * _CPW)], idx_v)
    pltpu.sync_copy(sb_hbm.at[pl.ds(2 * tbase, 2 * _CPW)], sb_v)

    def _start_gather(c):
        return pltpu.async_copy(
            ys_hbm.at[idx_v.at[pl.ds(c * 2 * _CCH, 2 * _CCH)]],
            rows2.at[c % 2], g_sem)

    gd = [None] * _CNCH
    wd = [None] * _CNCH
    gd[0] = _start_gather(0)
    for c in range(_CNCH):
        if c + 1 < _CNCH:
            gd[c + 1] = _start_gather(c + 1)
        gd[c].wait()
        if c >= 2:
            wd[c - 2].wait()
        rows_v = rows2.at[c % 2]
        out_v = out2.at[c % 2]

        @pl.loop(0, _CCH)
        def _tok(i):
            p = c * 2 * _CCH + 2 * i
            s0 = sb_v[p, :]
            s1 = sb_v[p + 1, :]
            for j in range(D_HALF // L):
                sl = pl.ds(j * L, L)
                a0 = rows_v[2 * i, sl]
                a1 = rows_v[2 * i + 1, sl]
                lo0 = plsc.bitcast(a0 << 16, _F32)
                lo1 = plsc.bitcast(a1 << 16, _F32)
                hi0 = plsc.bitcast(a0 & _HIMASK, _F32)
                hi1 = plsc.bitcast(a1 & _HIMASK, _F32)
                out_v[i, sl] = s0 * lo0 + s1 * lo1
                out_v[i, pl.ds(D_HALF + j * L, L)] = s0 * hi0 + s1 * hi1

        wd[c] = pltpu.async_copy(
            out_v, out_hbm.at[pl.ds(tbase + c * _CCH, _CCH)], w_sem)
    for c in range(max(0, _CNCH - 2), _CNCH):
        wd[c].wait()


def _combine(pos, sb, ys):
    return pl.kernel(
        _combine_body,
        out_type=jax.ShapeDtypeStruct((SEQ, D_MODEL), _F32),
        mesh=_sc_mesh(),
        compiler_params=pltpu.CompilerParams(needs_layout_passes=False),
        scratch_types=[
            pltpu.VMEM((2 * _CPW,), _I32),
            pltpu.VMEM((2 * _CPW, L), _F32),
            pltpu.VMEM((2, 2 * _CCH, D_HALF), _I32),
            pltpu.VMEM((2, _CCH, D_MODEL), _F32),
            pltpu.SemaphoreType.DMA,
            pltpu.SemaphoreType.DMA,
        ],
    )(pos, sb, ys)


def kernel(hidden_states, gate_w, w_gate, w_up, w_down):
    B, S, H = hidden_states.shape
    x = hidden_states.reshape(S, H)
    be = lax.bitcast_convert_type(
        x[:, :D_HALF].astype(jnp.bfloat16), jnp.uint16).astype(jnp.uint32)
    bo = lax.bitcast_convert_type(
        x[:, D_HALF:].astype(jnp.bfloat16), jnp.uint16).astype(jnp.uint32)
    x_p = lax.bitcast_convert_type(be | (bo << 16), _I32)

    logits, eids, rank, sb, meta = _router(x, gate_w)
    meta_flat = meta.reshape(128)
    pos, tok = _dispatch(eids.reshape(NPAIR), rank.reshape(NPAIR), meta_flat)
    xs = _gather(tok, x_p)
    ys = _ffn(meta_flat, xs, w_gate, w_up, w_down)
    out = _combine(pos, sb.reshape(NPAIR, L), ys)
    return out.reshape(B, S, H), logits


# speedup vs baseline: 1.2818x; 1.0042x over previous
import jax
import jax.numpy as jnp
from jax import lax
from jax.experimental import pallas as pl
from jax.experimental.pallas import tpu as pltpu
from jax.experimental.pallas import tpu_sc as plsc

NUM_EXPERTS = 8
D_MODEL = 2048
D_HALF = D_MODEL // 2
D_FF = 1024
SEQ = 2048
NPAIR = 2 * SEQ

TBLK = 256
NT = SEQ // TBLK
BT = 256
PMAX = NPAIR + NUM_EXPERTS * BT
GMAX = PMAX // BT
MOFF = 8
MNACT = MOFF + 64

NC, NS, L = 2, 16, 16
NW = NC * NS

_F32 = jnp.float32
_I32 = jnp.int32
_HIMASK = -65536


def _router_body(x_ref, gw_ref, logits_ref, eids_ref, rank_ref, sb_ref,
                 meta_ref, cnt_scr):
    t = pl.program_id(0)

    @pl.when(t == 0)
    def _init():
        cnt_scr[...] = jnp.zeros_like(cnt_scr)

    x = x_ref[...]
    logits = lax.dot_general(x, gw_ref[...], (((1,), (1,)), ((), ())),
                             preferred_element_type=_F32)
    logits_ref[...] = logits

    idx = lax.broadcasted_iota(_I32, (TBLK, NUM_EXPERTS), 1)
    m1 = jnp.max(logits, axis=-1, keepdims=True)
    e1 = jnp.min(jnp.where(logits == m1, idx, NUM_EXPERTS), axis=-1,
                 keepdims=True)
    lm = jnp.where(idx == e1, -jnp.inf, logits)
    m2 = jnp.max(lm, axis=-1, keepdims=True)
    e2 = jnp.min(jnp.where(lm == m2, idx, NUM_EXPERTS), axis=-1,
                 keepdims=True)
    w1 = 1.0 / (1.0 + jnp.exp(m2 - m1))
    w2 = 1.0 - w1

    eids_ref[...] = jnp.concatenate([e1, e2], axis=1)
    sb_ref[...] = jnp.concatenate(
        [jnp.broadcast_to(w1, (TBLK, L)), jnp.broadcast_to(w2, (TBLK, L))],
        axis=1)

    oh1 = jnp.where(idx == e1, 1.0, 0.0)
    oh2 = jnp.where(idx == e2, 1.0, 0.0)
    s_blk = oh1 + oh2

    r_io = lax.broadcasted_iota(_I32, (TBLK, TBLK), 0)
    c_io = lax.broadcasted_iota(_I32, (TBLK, TBLK), 1)
    lstrict = jnp.where(c_io < r_io, 1.0, 0.0)
    e_blk = lax.dot_general(lstrict, s_blk, (((1,), (0,)), ((), ())),
                            preferred_element_type=_F32) + cnt_scr[...]
    rank1 = jnp.sum(oh1 * e_blk, axis=-1, keepdims=True)
    rank2 = jnp.sum(oh2 * e_blk, axis=-1, keepdims=True)
    rank_ref[...] = jnp.concatenate([rank1, rank2], axis=1).astype(_I32)

    new_cnt = cnt_scr[...] + jnp.sum(s_blk, axis=0, keepdims=True)
    cnt_scr[...] = new_cnt

    @pl.when(t == NT - 1)
    def _meta():
        row_io = lax.broadcasted_iota(_I32, (8, 8), 0)
        col_io = lax.broadcasted_iota(_I32, (8, 8), 1)
        cnt_bc = jnp.broadcast_to(new_cnt, (8, 8))
        padded_bc = jnp.floor((cnt_bc + (BT - 1)) / BT) * BT
        cum_col = jnp.sum(jnp.where(col_io <= row_io, padded_bc, 0.0),
                          axis=1, keepdims=True)
        padded_col = jnp.sum(jnp.where(col_io == row_io, padded_bc, 0.0),
                             axis=1, keepdims=True)
        blocks_cum = cum_col / BT
        g_io = lax.broadcasted_iota(_I32, (8, 64), 1).astype(_F32)
        be_row = jnp.minimum(
            jnp.sum(jnp.where(g_io >= blocks_cum, 1.0, 0.0), axis=0,
                    keepdims=True), 7.0)
        padded_lbc = jnp.broadcast_to(padded_col, (8, 8))
        off_row = jnp.sum(jnp.where(row_io < col_io, padded_lbc, 0.0),
                          axis=0, keepdims=True)
        nact = jnp.sum(padded_col, axis=0, keepdims=True) / BT
        pad = jnp.zeros((1, 128 - MNACT - 1), _F32)
        meta_ref[...] = jnp.concatenate([off_row, be_row, nact, pad],
                                        axis=1).astype(_I32)


def _router(x, gate_w):
    return pl.pallas_call(
        _router_body,
        grid=(NT,),
        in_specs=[
            pl.BlockSpec((TBLK, D_MODEL), lambda t: (t, 0)),
            pl.BlockSpec((NUM_EXPERTS, D_MODEL), lambda t: (0, 0)),
        ],
        out_specs=[
            pl.BlockSpec((TBLK, NUM_EXPERTS), lambda t: (t, 0)),
            pl.BlockSpec((TBLK, 2), lambda t: (t, 0)),
            pl.BlockSpec((TBLK, 2), lambda t: (t, 0)),
            pl.BlockSpec((TBLK, 2 * L), lambda t: (t, 0)),
            pl.BlockSpec((1, 128), lambda t: (0, 0)),
        ],
        out_shape=[
            jax.ShapeDtypeStruct((SEQ, NUM_EXPERTS), _F32),
            jax.ShapeDtypeStruct((SEQ, 2), _I32),
            jax.ShapeDtypeStruct((SEQ, 2), _I32),
            jax.ShapeDtypeStruct((SEQ, 2 * L), _F32),
            jax.ShapeDtypeStruct((1, 128), _I32),
        ],
        scratch_shapes=[pltpu.VMEM((1, NUM_EXPERTS), _F32)],
        compiler_params=pltpu.CompilerParams(
            dimension_semantics=("arbitrary",)),
    )(x, gate_w)


def _sc_mesh():
    return plsc.VectorSubcoreMesh(core_axis_name="c", subcore_axis_name="s",
                                  num_cores=NC, num_subcores=NS)


def _dispatch_body(eids_hbm, rank_hbm, meta_hbm, pos_hbm, tok_hbm,
                   e_v, r_v, meta_v, pos_v, tok_v):
    wid = lax.axis_index("s") * NC + lax.axis_index("c")

    @pl.when(wid == 0)
    def _work():
        pltpu.sync_copy(eids_hbm, e_v)
        pltpu.sync_copy(rank_hbm, r_v)
        pltpu.sync_copy(meta_hbm, meta_v)

        @pl.loop(0, PMAX // L)
        def _zero(i):
            tok_v[pl.ds(i * L, L)] = jnp.zeros((L,), _I32)

        @pl.loop(0, NPAIR // L)
        def _scan(i):
            sl = pl.ds(i * L, L)
            e = e_v[sl]
            r = r_v[sl]
            off_e = plsc.load_gather(meta_v, [e])
            p = off_e + r
            pos_v[sl] = p
            tok = (i * L + lax.iota(_I32, L)) >> 1
            plsc.store_scatter(tok_v, [p], tok)

        pltpu.sync_copy(pos_v, pos_hbm)
        pltpu.sync_copy(tok_v, tok_hbm)


def _dispatch(eids, rank, meta):
    return pl.kernel(
        _dispatch_body,
        out_type=[
            jax.ShapeDtypeStruct((NPAIR,), _I32),
            jax.ShapeDtypeStruct((PMAX,), _I32),
        ],
        mesh=_sc_mesh(),
        compiler_params=pltpu.CompilerParams(needs_layout_passes=False),
        scratch_types=[
            pltpu.VMEM((NPAIR,), _I32),
            pltpu.VMEM((NPAIR,), _I32),
            pltpu.VMEM((128,), _I32),
            pltpu.VMEM((NPAIR,), _I32),
            pltpu.VMEM((PMAX,), _I32),
        ],
    )(eids, rank, meta)


_GCH = 32
_GPW = PMAX // NW
_GNCH = _GPW // _GCH
_GRING = 3


def _gather_body(tok_hbm, x_hbm, xs_hbm, idx_v, rows_r, g_sem, w_sem):
    wid = lax.axis_index("s") * NC + lax.axis_index("c")
    wbase = wid * _GPW
    pltpu.sync_copy(tok_hbm.at[pl.ds(wbase, _GPW)], idx_v)

    def _start_gather(c):
        return pltpu.async_copy(
            x_hbm.at[idx_v.at[pl.ds(c * _GCH, _GCH)]],
            rows_r.at[c % _GRING], g_sem)

    gd = [None] * _GNCH
    wd = [None] * _GNCH
    gd[0] = _start_gather(0)
    for c in range(_GNCH):
        if c + 1 < _GNCH:
            if c + 1 >= _GRING:
                wd[c + 1 - _GRING].wait()
            gd[c + 1] = _start_gather(c + 1)
        gd[c].wait()
        wd[c] = pltpu.async_copy(
            rows_r.at[c % _GRING], xs_hbm.at[pl.ds(wbase + c * _GCH, _GCH)],
            w_sem)
    for c in range(max(0, _GNCH - _GRING), _GNCH):
        wd[c].wait()


def _gather(tok, x_p):
    return pl.kernel(
        _gather_body,
        out_type=jax.ShapeDtypeStruct((PMAX, D_HALF), _I32),
        mesh=_sc_mesh(),
        compiler_params=pltpu.CompilerParams(needs_layout_passes=False),
        scratch_types=[
            pltpu.VMEM((_GPW,), _I32),
            pltpu.VMEM((_GRING, _GCH, D_HALF), _I32),
            pltpu.SemaphoreType.DMA,
            pltpu.SemaphoreType.DMA,
        ],
    )(tok, x_p)


def _ffn_body(meta_sref, xs_ref, wg_ref, wu_ref, wd_ref, ys_ref):
    g = pl.program_id(0)

    @pl.when(g < meta_sref[MNACT])
    def _active():
        xs = xs_ref[...]
        xlo = lax.bitcast_convert_type(xs << 16, _F32).astype(jnp.bfloat16)
        xhi = lax.bitcast_convert_type(xs & _HIMASK, _F32).astype(jnp.bfloat16)
        wg = wg_ref[0].astype(jnp.bfloat16)
        wu = wu_ref[0].astype(jnp.bfloat16)
        wd = wd_ref[0].astype(jnp.bfloat16)
        dims = (((1,), (1,)), ((), ()))
        gg = (lax.dot_general(xlo, wg[:, :D_HALF], dims,
                              preferred_element_type=_F32)
              + lax.dot_general(xhi, wg[:, D_HALF:], dims,
                                preferred_element_type=_F32))
        uu = (lax.dot_general(xlo, wu[:, :D_HALF], dims,
                              preferred_element_type=_F32)
              + lax.dot_general(xhi, wu[:, D_HALF:], dims,
                                preferred_element_type=_F32))
        h = (gg * jax.nn.sigmoid(gg) * uu).astype(jnp.bfloat16)
        y = lax.dot_general(h, wd, dims, preferred_element_type=_F32)
        ylo = lax.bitcast_convert_type(
            y[:, :D_HALF].astype(jnp.bfloat16), jnp.uint16).astype(jnp.uint32)
        yhi = lax.bitcast_convert_type(
            y[:, D_HALF:].astype(jnp.bfloat16), jnp.uint16).astype(jnp.uint32)
        ys_ref[...] = lax.bitcast_convert_type(ylo | (yhi << 16), _I32)


def _gclamp(g, m):
    return jnp.minimum(g, m[MNACT] - 1)


def _ffn(meta_flat, xs, wg, wu, wd):
    grid_spec = pltpu.PrefetchScalarGridSpec(
        num_scalar_prefetch=1,
        grid=(GMAX,),
        in_specs=[
            pl.BlockSpec((BT, D_HALF), lambda g, m: (_gclamp(g, m), 0)),
            pl.BlockSpec((1, D_FF, D_MODEL),
                         lambda g, m: (m[MOFF + _gclamp(g, m)], 0, 0)),
            pl.BlockSpec((1, D_FF, D_MODEL),
                         lambda g, m: (m[MOFF + _gclamp(g, m)], 0, 0)),
            pl.BlockSpec((1, D_MODEL, D_FF),
                         lambda g, m: (m[MOFF + _gclamp(g, m)], 0, 0)),
        ],
        out_specs=pl.BlockSpec((BT, D_HALF), lambda g, m: (_gclamp(g, m), 0)),
    )
    return pl.pallas_call(
        _ffn_body,
        grid_spec=grid_spec,
        out_shape=jax.ShapeDtypeStruct((PMAX, D_HALF), _I32),
        compiler_params=pltpu.CompilerParams(
            dimension_semantics=("arbitrary",)),
    )(meta_flat, xs, wg, wu, wd)


_CCH = 8
_CPW = SEQ // NW
_CNCH = _CPW // _CCH


def _combine_body(pos_hbm, sb_hbm, ys_hbm, out_hbm,
                  idx_v, sb_v, rows2, out2, g_sem, w_sem):
    wid = lax.axis_index("s") * NC + lax.axis_index("c")
    tbase = wid * _CPW
    pltpu.sync_copy(pos_hbm.at[pl.ds(2 * tbase, 2 * _CPW)], idx_v)
    pltpu.sync_copy(sb_hbm.at[pl.ds(2 * tbase, 2 * _CPW)], sb_v)

    def _start_gather(c):
        return pltpu.async_copy(
            ys_hbm.at[idx_v.at[pl.ds(c * 2 * _CCH, 2 * _CCH)]],
            rows2.at[c % 2], g_sem)

    gd = [None] * _CNCH
    wd = [None] * _CNCH
    gd[0] = _start_gather(0)
    for c in range(_CNCH):
        if c + 1 < _CNCH:
            gd[c + 1] = _start_gather(c + 1)
        gd[c].wait()
        if c >= 2:
            wd[c - 2].wait()
        rows_v = rows2.at[c % 2]
        out_v = out2.at[c % 2]

        @pl.loop(0, _CCH)
        def _tok(i):
            p = c * 2 * _CCH + 2 * i
            s0 = sb_v[p, :]
            s1 = sb_v[p + 1, :]
            for j in range(D_HALF // L):
                sl = pl.ds(j * L, L)
                a0 = rows_v[2 * i, sl]
                a1 = rows_v[2 * i + 1, sl]
                lo0 = plsc.bitcast(a0 << 16, _F32)
                lo1 = plsc.bitcast(a1 << 16, _F32)
                hi0 = plsc.bitcast(a0 & _HIMASK, _F32)
                hi1 = plsc.bitcast(a1 & _HIMASK, _F32)
                out_v[i, sl] = s0 * lo0 + s1 * lo1
                out_v[i, pl.ds(D_HALF + j * L, L)] = s0 * hi0 + s1 * hi1

        wd[c] = pltpu.async_copy(
            out_v, out_hbm.at[pl.ds(tbase + c * _CCH, _CCH)], w_sem)
    for c in range(max(0, _CNCH - 2), _CNCH):
        wd[c].wait()


def _combine(pos, sb, ys):
    return pl.kernel(
        _combine_body,
        out_type=jax.ShapeDtypeStruct((SEQ, D_MODEL), _F32),
        mesh=_sc_mesh(),
        compiler_params=pltpu.CompilerParams(needs_layout_passes=False),
        scratch_types=[
            pltpu.VMEM((2 * _CPW,), _I32),
            pltpu.VMEM((2 * _CPW, L), _F32),
            pltpu.VMEM((2, 2 * _CCH, D_HALF), _I32),
            pltpu.VMEM((2, _CCH, D_MODEL), _F32),
            pltpu.SemaphoreType.DMA,
            pltpu.SemaphoreType.DMA,
        ],
    )(pos, sb, ys)


def kernel(hidden_states, gate_w, w_gate, w_up, w_down):
    B, S, H = hidden_states.shape
    x = hidden_states.reshape(S, H)
    be = lax.bitcast_convert_type(
        x[:, :D_HALF].astype(jnp.bfloat16), jnp.uint16).astype(jnp.uint32)
    bo = lax.bitcast_convert_type(
        x[:, D_HALF:].astype(jnp.bfloat16), jnp.uint16).astype(jnp.uint32)
    x_p = lax.bitcast_convert_type(be | (bo << 16), _I32)

    logits, eids, rank, sb, meta = _router(x, gate_w)
    meta_flat = meta.reshape(128)
    pos, tok = _dispatch(eids.reshape(NPAIR), rank.reshape(NPAIR), meta_flat)
    xs = _gather(tok, x_p)
    ys = _ffn(meta_flat, xs, w_gate, w_up, w_down)
    out = _combine(pos, sb.reshape(NPAIR, L), ys)
    return out.reshape(B, S, H), logits


# speedup vs baseline: 1.2826x; 1.0006x over previous
import jax
import jax.numpy as jnp
from jax import lax
from jax.experimental import pallas as pl
from jax.experimental.pallas import tpu as pltpu
from jax.experimental.pallas import tpu_sc as plsc

NUM_EXPERTS = 8
D_MODEL = 2048
D_HALF = D_MODEL // 2
D_FF = 1024
SEQ = 2048
NPAIR = 2 * SEQ

TBLK = 256
NT = SEQ // TBLK
BT = 256
PMAX = NPAIR + NUM_EXPERTS * BT
GMAX = PMAX // BT
MOFF = 8
MNACT = MOFF + 64

NC, NS, L = 2, 16, 16
NW = NC * NS

_F32 = jnp.float32
_I32 = jnp.int32
_HIMASK = -65536


def _router_body(x_ref, gw_ref, logits_ref, eids_ref, rank_ref, sb_ref,
                 meta_ref, cnt_scr):
    t = pl.program_id(0)

    @pl.when(t == 0)
    def _init():
        cnt_scr[...] = jnp.zeros_like(cnt_scr)

    x = x_ref[...]
    logits = lax.dot_general(x, gw_ref[...], (((1,), (1,)), ((), ())),
                             preferred_element_type=_F32)
    logits_ref[...] = logits

    idx = lax.broadcasted_iota(_I32, (TBLK, NUM_EXPERTS), 1)
    m1 = jnp.max(logits, axis=-1, keepdims=True)
    e1 = jnp.min(jnp.where(logits == m1, idx, NUM_EXPERTS), axis=-1,
                 keepdims=True)
    lm = jnp.where(idx == e1, -jnp.inf, logits)
    m2 = jnp.max(lm, axis=-1, keepdims=True)
    e2 = jnp.min(jnp.where(lm == m2, idx, NUM_EXPERTS), axis=-1,
                 keepdims=True)
    w1 = 1.0 / (1.0 + jnp.exp(m2 - m1))
    w2 = 1.0 - w1

    eids_ref[...] = jnp.concatenate([e1, e2], axis=1)
    sb_ref[...] = jnp.concatenate(
        [jnp.broadcast_to(w1, (TBLK, L)), jnp.broadcast_to(w2, (TBLK, L))],
        axis=1)

    oh1 = jnp.where(idx == e1, 1.0, 0.0)
    oh2 = jnp.where(idx == e2, 1.0, 0.0)
    s_blk = oh1 + oh2

    r_io = lax.broadcasted_iota(_I32, (TBLK, TBLK), 0)
    c_io = lax.broadcasted_iota(_I32, (TBLK, TBLK), 1)
    lstrict = jnp.where(c_io < r_io, 1.0, 0.0)
    e_blk = lax.dot_general(lstrict, s_blk, (((1,), (0,)), ((), ())),
                            preferred_element_type=_F32) + cnt_scr[...]
    rank1 = jnp.sum(oh1 * e_blk, axis=-1, keepdims=True)
    rank2 = jnp.sum(oh2 * e_blk, axis=-1, keepdims=True)
    rank_ref[...] = jnp.concatenate([rank1, rank2], axis=1).astype(_I32)

    new_cnt = cnt_scr[...] + jnp.sum(s_blk, axis=0, keepdims=True)
    cnt_scr[...] = new_cnt

    @pl.when(t == NT - 1)
    def _meta():
        row_io = lax.broadcasted_iota(_I32, (8, 8), 0)
        col_io = lax.broadcasted_iota(_I32, (8, 8), 1)
        cnt_bc = jnp.broadcast_to(new_cnt, (8, 8))
        padded_bc = jnp.floor((cnt_bc + (BT - 1)) / BT) * BT
        cum_col = jnp.sum(jnp.where(col_io <= row_io, padded_bc, 0.0),
                          axis=1, keepdims=True)
        padded_col = jnp.sum(jnp.where(col_io == row_io, padded_bc, 0.0),
                             axis=1, keepdims=True)
        blocks_cum = cum_col / BT
        g_io = lax.broadcasted_iota(_I32, (8, 64), 1).astype(_F32)
        be_row = jnp.minimum(
            jnp.sum(jnp.where(g_io >= blocks_cum, 1.0, 0.0), axis=0,
                    keepdims=True), 7.0)
        padded_lbc = jnp.broadcast_to(padded_col, (8, 8))
        off_row = jnp.sum(jnp.where(row_io < col_io, padded_lbc, 0.0),
                          axis=0, keepdims=True)
        nact = jnp.sum(padded_col, axis=0, keepdims=True) / BT
        pad = jnp.zeros((1, 128 - MNACT - 1), _F32)
        meta_ref[...] = jnp.concatenate([off_row, be_row, nact, pad],
                                        axis=1).astype(_I32)


def _router(x, gate_w):
    return pl.pallas_call(
        _router_body,
        grid=(NT,),
        in_specs=[
            pl.BlockSpec((TBLK, D_MODEL), lambda t: (t, 0)),
            pl.BlockSpec((NUM_EXPERTS, D_MODEL), lambda t: (0, 0)),
        ],
        out_specs=[
            pl.BlockSpec((TBLK, NUM_EXPERTS), lambda t: (t, 0)),
            pl.BlockSpec((TBLK, 2), lambda t: (t, 0)),
            pl.BlockSpec((TBLK, 2), lambda t: (t, 0)),
            pl.BlockSpec((TBLK, 2 * L), lambda t: (t, 0)),
            pl.BlockSpec((1, 128), lambda t: (0, 0)),
        ],
        out_shape=[
            jax.ShapeDtypeStruct((SEQ, NUM_EXPERTS), _F32),
            jax.ShapeDtypeStruct((SEQ, 2), _I32),
            jax.ShapeDtypeStruct((SEQ, 2), _I32),
            jax.ShapeDtypeStruct((SEQ, 2 * L), _F32),
            jax.ShapeDtypeStruct((1, 128), _I32),
        ],
        scratch_shapes=[pltpu.VMEM((1, NUM_EXPERTS), _F32)],
        compiler_params=pltpu.CompilerParams(
            dimension_semantics=("arbitrary",)),
    )(x, gate_w)


def _sc_mesh():
    return plsc.VectorSubcoreMesh(core_axis_name="c", subcore_axis_name="s",
                                  num_cores=NC, num_subcores=NS)


def _dispatch_body(eids_hbm, rank_hbm, meta_hbm, pos_hbm, tok_hbm,
                   e_v, r_v, meta_v, pos_v, tok_v):
    wid = lax.axis_index("s") * NC + lax.axis_index("c")

    @pl.when(wid == 0)
    def _work():
        pltpu.sync_copy(eids_hbm, e_v)
        pltpu.sync_copy(rank_hbm, r_v)
        pltpu.sync_copy(meta_hbm, meta_v)

        @pl.loop(0, PMAX // L)
        def _zero(i):
            tok_v[pl.ds(i * L, L)] = jnp.zeros((L,), _I32)

        @pl.loop(0, NPAIR // L)
        def _scan(i):
            sl = pl.ds(i * L, L)
            e = e_v[sl]
            r = r_v[sl]
            off_e = plsc.load_gather(meta_v, [e])
            p = off_e + r
            pos_v[sl] = p
            tok = (i * L + lax.iota(_I32, L)) >> 1
            plsc.store_scatter(tok_v, [p], tok)

        pltpu.sync_copy(pos_v, pos_hbm)
        pltpu.sync_copy(tok_v, tok_hbm)


def _dispatch(eids, rank, meta):
    return pl.kernel(
        _dispatch_body,
        out_type=[
            jax.ShapeDtypeStruct((NPAIR,), _I32),
            jax.ShapeDtypeStruct((PMAX,), _I32),
        ],
        mesh=_sc_mesh(),
        compiler_params=pltpu.CompilerParams(needs_layout_passes=False),
        scratch_types=[
            pltpu.VMEM((NPAIR,), _I32),
            pltpu.VMEM((NPAIR,), _I32),
            pltpu.VMEM((128,), _I32),
            pltpu.VMEM((NPAIR,), _I32),
            pltpu.VMEM((PMAX,), _I32),
        ],
    )(eids, rank, meta)


_GCH = 24
_GPW = PMAX // NW
_GNCH = _GPW // _GCH
_GRING = 4


def _gather_body(tok_hbm, x_hbm, xs_hbm, idx_v, rows_r, g_sem, w_sem):
    wid = lax.axis_index("s") * NC + lax.axis_index("c")
    wbase = wid * _GPW
    pltpu.sync_copy(tok_hbm.at[pl.ds(wbase, _GPW)], idx_v)

    def _start_gather(c):
        return pltpu.async_copy(
            x_hbm.at[idx_v.at[pl.ds(c * _GCH, _GCH)]],
            rows_r.at[c % _GRING], g_sem)

    gd = [None] * _GNCH
    wd = [None] * _GNCH
    gd[0] = _start_gather(0)
    gd[1] = _start_gather(1)
    for c in range(_GNCH):
        if c + 2 < _GNCH:
            if c + 2 >= _GRING:
                wd[c + 2 - _GRING].wait()
            gd[c + 2] = _start_gather(c + 2)
        gd[c].wait()
        wd[c] = pltpu.async_copy(
            rows_r.at[c % _GRING], xs_hbm.at[pl.ds(wbase + c * _GCH, _GCH)],
            w_sem)
    for c in range(max(0, _GNCH - _GRING), _GNCH):
        wd[c].wait()


def _gather(tok, x_p):
    return pl.kernel(
        _gather_body,
        out_type=jax.ShapeDtypeStruct((PMAX, D_HALF), _I32),
        mesh=_sc_mesh(),
        compiler_params=pltpu.CompilerParams(needs_layout_passes=False),
        scratch_types=[
            pltpu.VMEM((_GPW,), _I32),
            pltpu.VMEM((_GRING, _GCH, D_HALF), _I32),
            pltpu.SemaphoreType.DMA,
            pltpu.SemaphoreType.DMA,
        ],
    )(tok, x_p)


def _ffn_body(meta_sref, xs_ref, wg_ref, wu_ref, wd_ref, ys_ref):
    g = pl.program_id(0)

    @pl.when(g < meta_sref[MNACT])
    def _active():
        xs = xs_ref[...]
        xlo = lax.bitcast_convert_type(xs << 16, _F32).astype(jnp.bfloat16)
        xhi = lax.bitcast_convert_type(xs & _HIMASK, _F32).astype(jnp.bfloat16)
        wg = wg_ref[0].astype(jnp.bfloat16)
        wu = wu_ref[0].astype(jnp.bfloat16)
        wd = wd_ref[0].astype(jnp.bfloat16)
        dims = (((1,), (1,)), ((), ()))
        gg = (lax.dot_general(xlo, wg[:, :D_HALF], dims,
                              preferred_element_type=_F32)
              + lax.dot_general(xhi, wg[:, D_HALF:], dims,
                                preferred_element_type=_F32))
        uu = (lax.dot_general(xlo, wu[:, :D_HALF], dims,
                              preferred_element_type=_F32)
              + lax.dot_general(xhi, wu[:, D_HALF:], dims,
                                preferred_element_type=_F32))
        h = (gg * jax.nn.sigmoid(gg) * uu).astype(jnp.bfloat16)
        y = lax.dot_general(h, wd, dims, preferred_element_type=_F32)
        ylo = lax.bitcast_convert_type(
            y[:, :D_HALF].astype(jnp.bfloat16), jnp.uint16).astype(jnp.uint32)
        yhi = lax.bitcast_convert_type(
            y[:, D_HALF:].astype(jnp.bfloat16), jnp.uint16).astype(jnp.uint32)
        ys_ref[...] = lax.bitcast_convert_type(ylo | (yhi << 16), _I32)


def _gclamp(g, m):
    return jnp.minimum(g, m[MNACT] - 1)


def _ffn(meta_flat, xs, wg, wu, wd):
    grid_spec = pltpu.PrefetchScalarGridSpec(
        num_scalar_prefetch=1,
        grid=(GMAX,),
        in_specs=[
            pl.BlockSpec((BT, D_HALF), lambda g, m: (_gclamp(g, m), 0)),
            pl.BlockSpec((1, D_FF, D_MODEL),
                         lambda g, m: (m[MOFF + _gclamp(g, m)], 0, 0)),
            pl.BlockSpec((1, D_FF, D_MODEL),
                         lambda g, m: (m[MOFF + _gclamp(g, m)], 0, 0)),
            pl.BlockSpec((1, D_MODEL, D_FF),
                         lambda g, m: (m[MOFF + _gclamp(g, m)], 0, 0)),
        ],
        out_specs=pl.BlockSpec((BT, D_HALF), lambda g, m: (_gclamp(g, m), 0)),
    )
    return pl.pallas_call(
        _ffn_body,
        grid_spec=grid_spec,
        out_shape=jax.ShapeDtypeStruct((PMAX, D_HALF), _I32),
        compiler_params=pltpu.CompilerParams(
            dimension_semantics=("arbitrary",)),
    )(meta_flat, xs, wg, wu, wd)


_CCH = 8
_CPW = SEQ // NW
_CNCH = _CPW // _CCH


def _combine_body(pos_hbm, sb_hbm, ys_hbm, out_hbm,
                  idx_v, sb_v, rows2, out2, g_sem, w_sem):
    wid = lax.axis_index("s") * NC + lax.axis_index("c")
    tbase = wid * _CPW
    pltpu.sync_copy(pos_hbm.at[pl.ds(2 * tbase, 2 * _CPW)], idx_v)
    pltpu.sync_copy(sb_hbm.at[pl.ds(2 * tbase, 2 * _CPW)], sb_v)

    def _start_gather(c):
        return pltpu.async_copy(
            ys_hbm.at[idx_v.at[pl.ds(c * 2 * _CCH, 2 * _CCH)]],
            rows2.at[c % 2], g_sem)

    gd = [None] * _CNCH
    wd = [None] * _CNCH
    gd[0] = _start_gather(0)
    for c in range(_CNCH):
        if c + 1 < _CNCH:
            gd[c + 1] = _start_gather(c + 1)
        gd[c].wait()
        if c >= 2:
            wd[c - 2].wait()
        rows_v = rows2.at[c % 2]
        out_v = out2.at[c % 2]

        @pl.loop(0, _CCH)
        def _tok(i):
            p = c * 2 * _CCH + 2 * i
            s0 = sb_v[p, :]
            s1 = sb_v[p + 1, :]
            for j in range(D_HALF // L):
                sl = pl.ds(j * L, L)
                a0 = rows_v[2 * i, sl]
                a1 = rows_v[2 * i + 1, sl]
                lo0 = plsc.bitcast(a0 << 16, _F32)
                lo1 = plsc.bitcast(a1 << 16, _F32)
                hi0 = plsc.bitcast(a0 & _HIMASK, _F32)
                hi1 = plsc.bitcast(a1 & _HIMASK, _F32)
                out_v[i, sl] = s0 * lo0 + s1 * lo1
                out_v[i, pl.ds(D_HALF + j * L, L)] = s0 * hi0 + s1 * hi1

        wd[c] = pltpu.async_copy(
            out_v, out_hbm.at[pl.ds(tbase + c * _CCH, _CCH)], w_sem)
    for c in range(max(0, _CNCH - 2), _CNCH):
        wd[c].wait()


def _combine(pos, sb, ys):
    return pl.kernel(
        _combine_body,
        out_type=jax.ShapeDtypeStruct((SEQ, D_MODEL), _F32),
        mesh=_sc_mesh(),
        compiler_params=pltpu.CompilerParams(needs_layout_passes=False),
        scratch_types=[
            pltpu.VMEM((2 * _CPW,), _I32),
            pltpu.VMEM((2 * _CPW, L), _F32),
            pltpu.VMEM((2, 2 * _CCH, D_HALF), _I32),
            pltpu.VMEM((2, _CCH, D_MODEL), _F32),
            pltpu.SemaphoreType.DMA,
            pltpu.SemaphoreType.DMA,
        ],
    )(pos, sb, ys)


def kernel(hidden_states, gate_w, w_gate, w_up, w_down):
    B, S, H = hidden_states.shape
    x = hidden_states.reshape(S, H)
    be = lax.bitcast_convert_type(
        x[:, :D_HALF].astype(jnp.bfloat16), jnp.uint16).astype(jnp.uint32)
    bo = lax.bitcast_convert_type(
        x[:, D_HALF:].astype(jnp.bfloat16), jnp.uint16).astype(jnp.uint32)
    x_p = lax.bitcast_convert_type(be | (bo << 16), _I32)

    logits, eids, rank, sb, meta = _router(x, gate_w)
    meta_flat = meta.reshape(128)
    pos, tok = _dispatch(eids.reshape(NPAIR), rank.reshape(NPAIR), meta_flat)
    xs = _gather(tok, x_p)
    ys = _ffn(meta_flat, xs, w_gate, w_up, w_down)
    out = _combine(pos, sb.reshape(NPAIR, L), ys)
    return out.reshape(B, S, H), logits
